# baseline (speedup 1.0000x reference)
import functools
import jax
import jax.numpy as jnp
from jax import lax
from jax.experimental import pallas as pl
from jax.experimental.pallas import tpu as pltpu
from jax.experimental.pallas import tpu_sc as plsc

B = 2; N = 10000; E = 320000; DF = 128; DE = 4; HID = 64; H = 4; DH = 16
NT = B * N
EC = B * E
SW = 16
FW = 16
A1W = 32
A2W = 64

EP = 655360
NTP = 20480
RPT = NTP // 16

_SC_PARAMS = pltpu.CompilerParams(needs_layout_passes=False,
                                  use_tc_tiling_on_sc=False)


def _prologue_body(x_ref, w_ref, as_ref, ad_ref, q_ref, s_ref, p_ref):
    h = jnp.dot(x_ref[...], w_ref[...], preferred_element_type=jnp.float32)
    ssrc = jnp.dot(h, as_ref[...], preferred_element_type=jnp.float32)
    sdst = jnp.dot(h, ad_ref[...], preferred_element_type=jnp.float32)
    z12 = jnp.zeros((h.shape[0], 12), jnp.float32)
    q_ref[...] = h
    s_ref[...] = jnp.concatenate([ssrc, z12], axis=1)
    p_ref[...] = jnp.concatenate([sdst, z12], axis=1)


def _make_qsp(x, w, a_s, a_d):
    blk = 1000
    return pl.pallas_call(
        _prologue_body,
        grid=(NT // blk,),
        in_specs=[
            pl.BlockSpec((blk, x.shape[1]), lambda i: (i, 0)),
            pl.BlockSpec(w.shape, lambda i: (0, 0)),
            pl.BlockSpec(a_s.shape, lambda i: (0, 0)),
            pl.BlockSpec(a_d.shape, lambda i: (0, 0)),
        ],
        out_specs=[
            pl.BlockSpec((blk, HID), lambda i: (i, 0)),
            pl.BlockSpec((blk, SW), lambda i: (i, 0)),
            pl.BlockSpec((blk, SW), lambda i: (i, 0)),
        ],
        out_shape=[
            jax.ShapeDtypeStruct((NT, HID), jnp.float32),
            jax.ShapeDtypeStruct((NT, SW), jnp.float32),
            jax.ShapeDtypeStruct((NT, SW), jnp.float32),
        ],
    )(x, w, a_s, a_d)


def _normalize(parts2, parts1, m1_ref, m2_ref, b_ref):
    acc2 = parts2[0] + parts2[1]
    acc1 = parts1[0] + parts1[1]
    num = acc2 + jnp.dot(acc1, m1_ref[...], preferred_element_type=jnp.float32)
    den = jnp.dot(acc1, m2_ref[...], preferred_element_type=jnp.float32)
    return jax.nn.relu(num / (den + 1e-16) + b_ref[...])


def _combine_body(p2_ref, p1_ref, m1_ref, m2_ref, b_ref, w2_ref, as_ref, ad_ref,
                  q_ref, s_ref, p_ref):
    hout = _normalize(p2_ref, p1_ref, m1_ref, m2_ref, b_ref)
    h2 = jnp.dot(hout, w2_ref[...], preferred_element_type=jnp.float32)
    ssrc = jnp.dot(h2, as_ref[...], preferred_element_type=jnp.float32)
    sdst = jnp.dot(h2, ad_ref[...], preferred_element_type=jnp.float32)
    z12 = jnp.zeros((h2.shape[0], 12), jnp.float32)
    q_ref[...] = h2
    s_ref[...] = jnp.concatenate([ssrc, z12], axis=1)
    p_ref[...] = jnp.concatenate([sdst, z12], axis=1)


def _combine_layer(parts2, parts1, m1, m2, b_row, w2, a_s, a_d):
    blk = 1000
    return pl.pallas_call(
        _combine_body,
        grid=(NT // blk,),
        in_specs=[
            pl.BlockSpec((2, blk, A2W), lambda i: (0, i, 0)),
            pl.BlockSpec((2, blk, A1W), lambda i: (0, i, 0)),
            pl.BlockSpec((A1W, HID), lambda i: (0, 0)),
            pl.BlockSpec((A1W, HID), lambda i: (0, 0)),
            pl.BlockSpec((1, HID), lambda i: (0, 0)),
            pl.BlockSpec((HID, HID), lambda i: (0, 0)),
            pl.BlockSpec((HID, H), lambda i: (0, 0)),
            pl.BlockSpec((HID, H), lambda i: (0, 0)),
        ],
        out_specs=[
            pl.BlockSpec((blk, HID), lambda i: (i, 0)),
            pl.BlockSpec((blk, SW), lambda i: (i, 0)),
            pl.BlockSpec((blk, SW), lambda i: (i, 0)),
        ],
        out_shape=[
            jax.ShapeDtypeStruct((NT, HID), jnp.float32),
            jax.ShapeDtypeStruct((NT, SW), jnp.float32),
            jax.ShapeDtypeStruct((NT, SW), jnp.float32),
        ],
    )(parts2, parts1, m1, m2, b_row, w2, a_s, a_d)


def _head_body(p2_ref, p1_ref, m1_ref, m2_ref, b_ref, rnn_ref, mask_ref,
               wz_ref, uz_ref, bz_ref, wr_ref, ur_ref, br_ref,
               wn_ref, un_ref, bn_ref, wv_ref, bv_ref,
               val_ref, hnew_ref, acc_ref):
    i = pl.program_id(0)
    hout = _normalize(p2_ref, p1_ref, m1_ref, m2_ref, b_ref)
    contrib = jnp.sum(hout, axis=0, keepdims=True) * (1.0 / N)

    @pl.when(i == 0)
    def _():
        acc_ref[...] = jnp.zeros_like(acc_ref)

    brow = i // (N // 1000)
    acc_ref[pl.ds(brow, 1), :] += contrib

    @pl.when(i == (NT // 1000) - 1)
    def _():
        pooled = acc_ref[...]
        hprev = rnn_ref[...] * mask_ref[...]
        z = jax.nn.sigmoid(jnp.dot(pooled, wz_ref[...], preferred_element_type=jnp.float32)
                           + jnp.dot(hprev, uz_ref[...], preferred_element_type=jnp.float32)
                           + bz_ref[...])
        r = jax.nn.sigmoid(jnp.dot(pooled, wr_ref[...], preferred_element_type=jnp.float32)
                           + jnp.dot(hprev, ur_ref[...], preferred_element_type=jnp.float32)
                           + br_ref[...])
        n = jnp.tanh(jnp.dot(pooled, wn_ref[...], preferred_element_type=jnp.float32)
                     + r * jnp.dot(hprev, un_ref[...], preferred_element_type=jnp.float32)
                     + bn_ref[...])
        hnew = (1.0 - z) * n + z * hprev
        hnew_ref[...] = hnew
        val_ref[...] = jnp.dot(hnew, wv_ref[...], preferred_element_type=jnp.float32) + bv_ref[...]


def _head(parts2, parts1, m1, m2, b_row, rnn, masks, wz, uz, bz, wr, ur, br,
          wn, un, bn, wv, bv):
    blk = 1000
    full = lambda s: pl.BlockSpec(s, lambda i: tuple(0 for _ in s))
    return pl.pallas_call(
        _head_body,
        grid=(NT // blk,),
        in_specs=[
            pl.BlockSpec((2, blk, A2W), lambda i: (0, i, 0)),
            pl.BlockSpec((2, blk, A1W), lambda i: (0, i, 0)),
            full((A1W, HID)), full((A1W, HID)), full((1, HID)),
            full((B, HID)), full((B, 1)),
            full((HID, HID)), full((HID, HID)), full((1, HID)),
            full((HID, HID)), full((HID, HID)), full((1, HID)),
            full((HID, HID)), full((HID, HID)), full((1, HID)),
            full((HID, 1)), full((1, 1)),
        ],
        out_specs=[full((B, 1)), full((B, HID))],
        out_shape=[
            jax.ShapeDtypeStruct((B, 1), jnp.float32),
            jax.ShapeDtypeStruct((B, HID), jnp.float32),
        ],
        scratch_shapes=[pltpu.VMEM((B, HID), jnp.float32)],
    )(parts2, parts1, m1, m2, b_row, rnn, masks, wz, uz, bz, wr, ur, br,
      wn, un, bn, wv, bv)


def _full16(v):
    return jnp.full((16,), v, jnp.int32)


CHE = 128
NCH = EP // (32 * CHE)


def _zero_acc(buf, acc, sid, width):
    zero16 = jnp.zeros((16,), jnp.float32)

    def _zrow(j, _):
        for cgrp in range(width // 16):
            buf[j, pl.ds(cgrp * 16, 16)] = zero16
        return _
    lax.fori_loop(0, CHE, _zrow, None)
    for i in range(RPT // CHE):
        pltpu.sync_copy(buf, acc.at[pl.ds(sid * RPT + i * CHE, CHE)])


def _sc_logit_pass(s, p, eap, cb, src2d, dst2d):
    mesh = plsc.VectorSubcoreMesh(core_axis_name="c", subcore_axis_name="s")

    @functools.partial(
        pl.kernel,
        out_type=[jax.ShapeDtypeStruct((2, NTP, A1W), jnp.float32),
                  jax.ShapeDtypeStruct((EP, 4), jnp.float32)],
        mesh=mesh,
        compiler_params=_SC_PARAMS,
        scratch_types=[
            [pltpu.VMEM((1, 128), jnp.int32) for _ in range(2)],
            [pltpu.VMEM((1, 128), jnp.int32) for _ in range(2)],
            [pltpu.VMEM((CHE, SW), jnp.float32) for _ in range(2)],
            [pltpu.VMEM((CHE, SW), jnp.float32) for _ in range(2)],
            pltpu.VMEM((CHE, 4), jnp.float32),
            pltpu.VMEM((16, 16), jnp.float32),
            pltpu.VMEM((CHE, A1W), jnp.float32),
            pltpu.VMEM((CHE, 4), jnp.float32),
            pltpu.VMEM_SHARED((NTP, A1W), jnp.float32),
            [pltpu.SemaphoreType.DMA for _ in range(2)],
        ],
    )
    def k(s_hbm, p_hbm, ea_hbm, cb_hbm, src_hbm, dst_hbm, out_hbm, w_hbm,
          srcv, dstv, sg, pg, eav, cbv, pay, wv, acc, sems):
        cid = lax.axis_index("c")
        sid = lax.axis_index("s")
        wid = sid * 2 + cid
        row0 = wid * NCH
        lanes = lax.iota(jnp.int32, 16)

        pltpu.sync_copy(cb_hbm, cbv)
        cr = [cbv[i, :] for i in range(16)]
        _zero_acc(pay, acc, sid, A1W)
        plsc.subcore_barrier()

        def _fire(gnext, b):
            @pl.when(gnext < NCH)
            def _():
                pltpu.sync_copy(src_hbm.at[pl.ds(row0 + gnext, 1)], srcv[b])
                pltpu.sync_copy(dst_hbm.at[pl.ds(row0 + gnext, 1)], dstv[b])
                pltpu.async_copy(s_hbm.at[srcv[b].at[0]], sg[b], sems[b])
                pltpu.async_copy(p_hbm.at[dstv[b].at[0]], pg[b], sems[b])

        def _half(g, b):
            _fire(g, b)
            pltpu.make_async_copy(s_hbm.at[srcv[b].at[0]], sg[b], sems[b]).wait()
            pltpu.make_async_copy(p_hbm.at[dstv[b].at[0]], pg[b], sems[b]).wait()
            ebase = (row0 + g) * 128
            pltpu.sync_copy(ea_hbm.at[pl.ds(ebase, CHE)], eav)

            def _grp(j, _):
                el = j * 16 + lanes
                keep = (ebase + el) < EC
                ea = [plsc.load_gather(eav, [el, _full16(d)]) for d in range(DE)]
                for h in range(H):
                    ssrc = plsc.load_gather(sg[b], [el, _full16(h)])
                    sdst = plsc.load_gather(pg[b], [el, _full16(h)])
                    et = (ea[0] * cr[h] + ea[1] * cr[4 + h]
                          + ea[2] * cr[8 + h] + ea[3] * cr[12 + h])
                    lg = ssrc + sdst + et
                    lg = jnp.maximum(lg, 0.2 * lg)
                    w = jnp.where(keep, jnp.exp(lg), 0.0)
                    plsc.store_scatter(pay, [el, _full16(h)], w)
                    plsc.store_scatter(wv, [el, _full16(h)], w)
                    for d in range(DE):
                        plsc.store_scatter(pay, [el, _full16(4 + h * DE + d)],
                                           w * ea[d])
                return _
            lax.fori_loop(0, CHE // 16, _grp, None)

            pltpu.sync_copy(wv, w_hbm.at[pl.ds(ebase, CHE)])
            pltpu.sync_copy(pay, acc.at[dstv[b].at[0]], add=True)

        def _pair(g0, _):
            _half(2 * g0, 0)
            _half(2 * g0 + 1, 1)
            return _
        lax.fori_loop(0, NCH // 2, _pair, None)

        plsc.subcore_barrier()
        pltpu.sync_copy(acc.at[pl.ds(sid * RPT, RPT)],
                        out_hbm.at[cid, pl.ds(sid * RPT, RPT)])

    return k(s, p, eap, cb, src2d, dst2d)


def _sc_message_pass(q, warr, src2d, dst2d):
    mesh = plsc.VectorSubcoreMesh(core_axis_name="c", subcore_axis_name="s")

    @functools.partial(
        pl.kernel,
        out_type=jax.ShapeDtypeStruct((2, NTP, A2W), jnp.float32),
        mesh=mesh,
        compiler_params=_SC_PARAMS,
        scratch_types=[
            [pltpu.VMEM((1, 128), jnp.int32) for _ in range(2)],
            pltpu.VMEM((1, 128), jnp.int32),
            pltpu.VMEM((CHE, 4), jnp.float32),
            [pltpu.VMEM((CHE, A2W), jnp.float32) for _ in range(2)],
            pltpu.VMEM((CHE, A2W), jnp.float32),
            pltpu.VMEM_SHARED((NTP, A2W), jnp.float32),
            [pltpu.SemaphoreType.DMA for _ in range(2)],
        ],
    )
    def k(q_hbm, w_hbm, src_hbm, dst_hbm, out_hbm,
          srcv, dstv, wv, qg, pay, acc, sems):
        cid = lax.axis_index("c")
        sid = lax.axis_index("s")
        wid = sid * 2 + cid
        row0 = wid * NCH
        lanes = lax.iota(jnp.int32, 16)

        _zero_acc(pay, acc, sid, A2W)
        plsc.subcore_barrier()

        def _fire(gnext, b):
            @pl.when(gnext < NCH)
            def _():
                pltpu.sync_copy(src_hbm.at[pl.ds(row0 + gnext, 1)], srcv[b])
                pltpu.async_copy(q_hbm.at[srcv[b].at[0]], qg[b], sems[b])

        def _half(g, b):
            _fire(g, b)
            pltpu.make_async_copy(q_hbm.at[srcv[b].at[0]], qg[b], sems[b]).wait()
            ebase = (row0 + g) * 128
            pltpu.sync_copy(w_hbm.at[pl.ds(ebase, CHE)], wv)
            pltpu.sync_copy(dst_hbm.at[pl.ds(row0 + g, 1)], dstv)

            def _grp(j, _):
                el = j * 16 + lanes
                for h in range(H):
                    w = plsc.load_gather(wv, [el, _full16(h)])
                    for kk in range(DH):
                        col = _full16(h * DH + kk)
                        hv = plsc.load_gather(qg[b], [el, col])
                        plsc.store_scatter(pay, [el, col], w * hv)
                return _
            lax.fori_loop(0, CHE // 16, _grp, None)

            pltpu.sync_copy(pay, acc.at[dstv.at[0]], add=True)

        def _pair(g0, _):
            _half(2 * g0, 0)
            _half(2 * g0 + 1, 1)
            return _
        lax.fori_loop(0, NCH // 2, _pair, None)

        plsc.subcore_barrier()
        pltpu.sync_copy(acc.at[pl.ds(sid * RPT, RPT)],
                        out_hbm.at[cid, pl.ds(sid * RPT, RPT)])

    return k(q, warr, src2d, dst2d)


def _blockdiag_scores(a):
    m = jnp.zeros((HID, H), jnp.float32)
    for h in range(H):
        m = m.at[h * DH:(h + 1) * DH, h].set(a[h])
    return m


def _combine_mats(we):
    m1 = jnp.zeros((A1W, HID), jnp.float32)
    m2 = jnp.zeros((A1W, HID), jnp.float32)
    for h in range(H):
        m1 = m1.at[4 + h * DE:4 + (h + 1) * DE, h * DH:(h + 1) * DH].set(
            we[:, h * DH:(h + 1) * DH])
        m2 = m2.at[h, h * DH:(h + 1) * DH].set(1.0)
    return m1, m2


def _eterm_coeff(we, a_e):
    c = jnp.stack([we[:, h * DH:(h + 1) * DH] @ a_e[h] for h in range(H)],
                  axis=1)
    return jnp.tile(c.reshape(16, 1), (1, 16))


def kernel(agent_id, bacth_nodes_feats, bacth_edge_index, bacth_edge_attr,
           rnn_states, masks, W1, We1, a_src1, a_dst1, a_e1, b1,
           W2, We2, a_src2, a_dst2, a_e2, b2,
           Wz, Uz, bz, Wr, Ur, br, Wn, Un, bn, Wv, bv):
    nodes = bacth_nodes_feats[:, 0].reshape(NT, DF)
    ei = bacth_edge_index[:, 0]
    eattr = bacth_edge_attr[:, 0].reshape(EC, DE)
    offs = (jnp.arange(B, dtype=jnp.int32) * N)[:, None]
    src = (ei[:, 0, :] + offs).reshape(EC)
    dst = (ei[:, 1, :] + offs).reshape(EC)
    src2d = jnp.pad(src, (0, EP - EC)).reshape(EP // 128, 128)
    dst2d = jnp.pad(dst, (0, EP - EC)).reshape(EP // 128, 128)
    eap = jnp.pad(eattr, ((0, EP - EC), (0, 0)))

    as1 = _blockdiag_scores(a_src1); ad1 = _blockdiag_scores(a_dst1)
    as2 = _blockdiag_scores(a_src2); ad2 = _blockdiag_scores(a_dst2)
    c1 = _eterm_coeff(We1, a_e1); c2 = _eterm_coeff(We2, a_e2)
    m1a, m2a = _combine_mats(We1)
    m1b, m2b = _combine_mats(We2)

    q1, s1, p1 = _make_qsp(nodes, W1, as1, ad1)
    parts1a, w1arr = _sc_logit_pass(s1, p1, eap, c1, src2d, dst2d)
    parts1b = _sc_message_pass(q1, w1arr, src2d, dst2d)
    q2, s2, p2 = _combine_layer(parts1b, parts1a, m1a, m2a, b1[None, :],
                                W2, as2, ad2)
    parts2a, w2arr = _sc_logit_pass(s2, p2, eap, c2, src2d, dst2d)
    parts2b = _sc_message_pass(q2, w2arr, src2d, dst2d)
    values, hnew = _head(parts2b, parts2a, m1b, m2b, b2[None, :],
                         rnn_states[:, 0], masks,
                         Wz, Uz, bz[None, :], Wr, Ur, br[None, :],
                         Wn, Un, bn[None, :], Wv, bv[None, :])
    return values, hnew[:, None, :]


# speedup vs baseline: 1.1909x; 1.1909x over previous
import functools
import jax
import jax.numpy as jnp
from jax import lax
from jax.experimental import pallas as pl
from jax.experimental.pallas import tpu as pltpu
from jax.experimental.pallas import tpu_sc as plsc

B = 2; N = 10000; E = 320000; DF = 128; DE = 4; HID = 64; H = 4; DH = 16
NT = B * N
EC = B * E
SW = 16
FW = 16
A1W = 32
A2W = 64

EP = 655360
NTP = 20480
RPT = NTP // 16

_SC_PARAMS = pltpu.CompilerParams(needs_layout_passes=False,
                                  use_tc_tiling_on_sc=False)


def _prologue_body(x_ref, w_ref, as_ref, ad_ref, q_ref, s_ref, p_ref):
    h = jnp.dot(x_ref[...], w_ref[...], preferred_element_type=jnp.float32)
    ssrc = jnp.dot(h, as_ref[...], preferred_element_type=jnp.float32)
    sdst = jnp.dot(h, ad_ref[...], preferred_element_type=jnp.float32)
    z12 = jnp.zeros((h.shape[0], 12), jnp.float32)
    q_ref[...] = h
    s_ref[...] = jnp.concatenate([ssrc, z12], axis=1)
    p_ref[...] = jnp.concatenate([sdst, z12], axis=1)


def _make_qsp(x, w, a_s, a_d):
    blk = 1000
    return pl.pallas_call(
        _prologue_body,
        grid=(NT // blk,),
        in_specs=[
            pl.BlockSpec((blk, x.shape[1]), lambda i: (i, 0)),
            pl.BlockSpec(w.shape, lambda i: (0, 0)),
            pl.BlockSpec(a_s.shape, lambda i: (0, 0)),
            pl.BlockSpec(a_d.shape, lambda i: (0, 0)),
        ],
        out_specs=[
            pl.BlockSpec((blk, HID), lambda i: (i, 0)),
            pl.BlockSpec((blk, SW), lambda i: (i, 0)),
            pl.BlockSpec((blk, SW), lambda i: (i, 0)),
        ],
        out_shape=[
            jax.ShapeDtypeStruct((NT, HID), jnp.float32),
            jax.ShapeDtypeStruct((NT, SW), jnp.float32),
            jax.ShapeDtypeStruct((NT, SW), jnp.float32),
        ],
    )(x, w, a_s, a_d)


def _normalize(parts2, parts1, m1_ref, m2_ref, b_ref):
    acc2 = parts2[0] + parts2[1]
    acc1 = parts1[0] + parts1[1]
    num = acc2 + jnp.dot(acc1, m1_ref[...], preferred_element_type=jnp.float32)
    den = jnp.dot(acc1, m2_ref[...], preferred_element_type=jnp.float32)
    return jax.nn.relu(num / (den + 1e-16) + b_ref[...])


def _combine_body(p2_ref, p1_ref, m1_ref, m2_ref, b_ref, w2_ref, as_ref, ad_ref,
                  q_ref, s_ref, p_ref):
    hout = _normalize(p2_ref, p1_ref, m1_ref, m2_ref, b_ref)
    h2 = jnp.dot(hout, w2_ref[...], preferred_element_type=jnp.float32)
    ssrc = jnp.dot(h2, as_ref[...], preferred_element_type=jnp.float32)
    sdst = jnp.dot(h2, ad_ref[...], preferred_element_type=jnp.float32)
    z12 = jnp.zeros((h2.shape[0], 12), jnp.float32)
    q_ref[...] = h2
    s_ref[...] = jnp.concatenate([ssrc, z12], axis=1)
    p_ref[...] = jnp.concatenate([sdst, z12], axis=1)


def _combine_layer(parts2, parts1, m1, m2, b_row, w2, a_s, a_d):
    blk = 1000
    return pl.pallas_call(
        _combine_body,
        grid=(NT // blk,),
        in_specs=[
            pl.BlockSpec((2, blk, A2W), lambda i: (0, i, 0)),
            pl.BlockSpec((2, blk, A1W), lambda i: (0, i, 0)),
            pl.BlockSpec((A1W, HID), lambda i: (0, 0)),
            pl.BlockSpec((A1W, HID), lambda i: (0, 0)),
            pl.BlockSpec((1, HID), lambda i: (0, 0)),
            pl.BlockSpec((HID, HID), lambda i: (0, 0)),
            pl.BlockSpec((HID, H), lambda i: (0, 0)),
            pl.BlockSpec((HID, H), lambda i: (0, 0)),
        ],
        out_specs=[
            pl.BlockSpec((blk, HID), lambda i: (i, 0)),
            pl.BlockSpec((blk, SW), lambda i: (i, 0)),
            pl.BlockSpec((blk, SW), lambda i: (i, 0)),
        ],
        out_shape=[
            jax.ShapeDtypeStruct((NT, HID), jnp.float32),
            jax.ShapeDtypeStruct((NT, SW), jnp.float32),
            jax.ShapeDtypeStruct((NT, SW), jnp.float32),
        ],
    )(parts2, parts1, m1, m2, b_row, w2, a_s, a_d)


def _head_body(p2_ref, p1_ref, m1_ref, m2_ref, b_ref, rnn_ref, mask_ref,
               wz_ref, uz_ref, bz_ref, wr_ref, ur_ref, br_ref,
               wn_ref, un_ref, bn_ref, wv_ref, bv_ref,
               val_ref, hnew_ref, acc_ref):
    i = pl.program_id(0)
    hout = _normalize(p2_ref, p1_ref, m1_ref, m2_ref, b_ref)
    contrib = jnp.sum(hout, axis=0, keepdims=True) * (1.0 / N)

    @pl.when(i == 0)
    def _():
        acc_ref[...] = jnp.zeros_like(acc_ref)

    brow = i // (N // 1000)
    acc_ref[pl.ds(brow, 1), :] += contrib

    @pl.when(i == (NT // 1000) - 1)
    def _():
        pooled = acc_ref[...]
        hprev = rnn_ref[...] * mask_ref[...]
        z = jax.nn.sigmoid(jnp.dot(pooled, wz_ref[...], preferred_element_type=jnp.float32)
                           + jnp.dot(hprev, uz_ref[...], preferred_element_type=jnp.float32)
                           + bz_ref[...])
        r = jax.nn.sigmoid(jnp.dot(pooled, wr_ref[...], preferred_element_type=jnp.float32)
                           + jnp.dot(hprev, ur_ref[...], preferred_element_type=jnp.float32)
                           + br_ref[...])
        n = jnp.tanh(jnp.dot(pooled, wn_ref[...], preferred_element_type=jnp.float32)
                     + r * jnp.dot(hprev, un_ref[...], preferred_element_type=jnp.float32)
                     + bn_ref[...])
        hnew = (1.0 - z) * n + z * hprev
        hnew_ref[...] = hnew
        val_ref[...] = jnp.dot(hnew, wv_ref[...], preferred_element_type=jnp.float32) + bv_ref[...]


def _head(parts2, parts1, m1, m2, b_row, rnn, masks, wz, uz, bz, wr, ur, br,
          wn, un, bn, wv, bv):
    blk = 1000
    full = lambda s: pl.BlockSpec(s, lambda i: tuple(0 for _ in s))
    return pl.pallas_call(
        _head_body,
        grid=(NT // blk,),
        in_specs=[
            pl.BlockSpec((2, blk, A2W), lambda i: (0, i, 0)),
            pl.BlockSpec((2, blk, A1W), lambda i: (0, i, 0)),
            full((A1W, HID)), full((A1W, HID)), full((1, HID)),
            full((B, HID)), full((B, 1)),
            full((HID, HID)), full((HID, HID)), full((1, HID)),
            full((HID, HID)), full((HID, HID)), full((1, HID)),
            full((HID, HID)), full((HID, HID)), full((1, HID)),
            full((HID, 1)), full((1, 1)),
        ],
        out_specs=[full((B, 1)), full((B, HID))],
        out_shape=[
            jax.ShapeDtypeStruct((B, 1), jnp.float32),
            jax.ShapeDtypeStruct((B, HID), jnp.float32),
        ],
        scratch_shapes=[pltpu.VMEM((B, HID), jnp.float32)],
    )(parts2, parts1, m1, m2, b_row, rnn, masks, wz, uz, bz, wr, ur, br,
      wn, un, bn, wv, bv)


def _full16(v):
    return jnp.full((16,), v, jnp.int32)


CHE = 128
NCH = EP // (32 * CHE)


def _zero_acc(buf, acc, sid, width):
    zero16 = jnp.zeros((16,), jnp.float32)

    def _zrow(j, _):
        for cgrp in range(width // 16):
            buf[j, pl.ds(cgrp * 16, 16)] = zero16
        return _
    lax.fori_loop(0, CHE, _zrow, None)
    for i in range(RPT // CHE):
        pltpu.sync_copy(buf, acc.at[pl.ds(sid * RPT + i * CHE, CHE)])


def _sc_logit_pass(s, p, eap, cb, src2d, dst2d):
    mesh = plsc.VectorSubcoreMesh(core_axis_name="c", subcore_axis_name="s")

    @functools.partial(
        pl.kernel,
        out_type=[jax.ShapeDtypeStruct((2, NTP, A1W), jnp.float32),
                  jax.ShapeDtypeStruct((EP, 4), jnp.float32)],
        mesh=mesh,
        compiler_params=_SC_PARAMS,
        scratch_types=[
            [pltpu.VMEM((1, 128), jnp.int32) for _ in range(2)],
            [pltpu.VMEM((1, 128), jnp.int32) for _ in range(2)],
            [pltpu.VMEM((1, 128), jnp.int32) for _ in range(2)],
            [pltpu.VMEM((CHE, SW), jnp.float32) for _ in range(2)],
            [pltpu.VMEM((CHE, SW), jnp.float32) for _ in range(2)],
            pltpu.VMEM((CHE, 4), jnp.float32),
            pltpu.VMEM((16, 16), jnp.float32),
            [pltpu.VMEM((CHE, A1W), jnp.float32) for _ in range(2)],
            pltpu.VMEM((CHE, 4), jnp.float32),
            pltpu.VMEM_SHARED((NTP, A1W), jnp.float32),
            [pltpu.SemaphoreType.DMA for _ in range(2)],
            [pltpu.SemaphoreType.DMA for _ in range(2)],
        ],
    )
    def k(s_hbm, p_hbm, ea_hbm, cb_hbm, src_hbm, dst_hbm, out_hbm, w_hbm,
          srcv, dgat, dsca, sg, pg, eav, cbv, pay, wv, acc, gsems, ssems):
        cid = lax.axis_index("c")
        sid = lax.axis_index("s")
        wid = sid * 2 + cid
        row0 = wid * NCH
        lanes = lax.iota(jnp.int32, 16)

        pltpu.sync_copy(cb_hbm, cbv)
        cr = [cbv[i, :] for i in range(16)]
        _zero_acc(pay[0], acc, sid, A1W)
        zero16 = jnp.zeros((16,), jnp.float32)

        def _z1(j, _):
            for cgrp in range(A1W // 16):
                pay[1][j, pl.ds(cgrp * 16, 16)] = zero16
            return _
        lax.fori_loop(0, CHE, _z1, None)
        plsc.subcore_barrier()

        def _fire(gnext, b):
            @pl.when(gnext < NCH)
            def _():
                pltpu.sync_copy(src_hbm.at[pl.ds(row0 + gnext, 1)], srcv[b])
                pltpu.sync_copy(dst_hbm.at[pl.ds(row0 + gnext, 1)], dgat[b])
                pltpu.async_copy(s_hbm.at[srcv[b].at[0]], sg[b], gsems[b])
                pltpu.async_copy(p_hbm.at[dgat[b].at[0]], pg[b], gsems[b])

        def _swait(b):
            pltpu.make_async_copy(pay[b], acc.at[dsca[b].at[0]], ssems[b]).wait()

        _fire(0, 0)

        def _half(g, b):
            @pl.when(g >= 2)
            def _():
                _swait(b)
            _fire(g + 1, 1 - b)
            pltpu.make_async_copy(s_hbm.at[srcv[b].at[0]], sg[b], gsems[b]).wait()
            pltpu.make_async_copy(p_hbm.at[dgat[b].at[0]], pg[b], gsems[b]).wait()
            ebase = (row0 + g) * 128
            pltpu.sync_copy(ea_hbm.at[pl.ds(ebase, CHE)], eav)
            pltpu.sync_copy(dst_hbm.at[pl.ds(row0 + g, 1)], dsca[b])

            def _grp(j, _):
                el = j * 16 + lanes
                keep = (ebase + el) < EC
                ea = [plsc.load_gather(eav, [el, _full16(d)]) for d in range(DE)]
                for h in range(H):
                    ssrc = plsc.load_gather(sg[b], [el, _full16(h)])
                    sdst = plsc.load_gather(pg[b], [el, _full16(h)])
                    et = (ea[0] * cr[h] + ea[1] * cr[4 + h]
                          + ea[2] * cr[8 + h] + ea[3] * cr[12 + h])
                    lg = ssrc + sdst + et
                    lg = jnp.maximum(lg, 0.2 * lg)
                    w = jnp.where(keep, jnp.exp(lg), 0.0)
                    plsc.store_scatter(pay[b], [el, _full16(h)], w)
                    plsc.store_scatter(wv, [el, _full16(h)], w)
                    for d in range(DE):
                        plsc.store_scatter(pay[b], [el, _full16(4 + h * DE + d)],
                                           w * ea[d])
                return _
            lax.fori_loop(0, CHE // 16, _grp, None)

            pltpu.sync_copy(wv, w_hbm.at[pl.ds(ebase, CHE)])
            pltpu.async_copy(pay[b], acc.at[dsca[b].at[0]], ssems[b], add=True)

        def _pair(g0, _):
            _half(2 * g0, 0)
            _half(2 * g0 + 1, 1)
            return _
        lax.fori_loop(0, NCH // 2, _pair, None)

        _swait(0)
        _swait(1)
        plsc.subcore_barrier()
        pltpu.sync_copy(acc.at[pl.ds(sid * RPT, RPT)],
                        out_hbm.at[cid, pl.ds(sid * RPT, RPT)])

    return k(s, p, eap, cb, src2d, dst2d)


def _sc_message_pass(q, warr, src2d, dst2d):
    mesh = plsc.VectorSubcoreMesh(core_axis_name="c", subcore_axis_name="s")

    @functools.partial(
        pl.kernel,
        out_type=jax.ShapeDtypeStruct((2, NTP, A2W), jnp.float32),
        mesh=mesh,
        compiler_params=_SC_PARAMS,
        scratch_types=[
            [pltpu.VMEM((1, 128), jnp.int32) for _ in range(2)],
            [pltpu.VMEM((1, 128), jnp.int32) for _ in range(2)],
            pltpu.VMEM((CHE, 4), jnp.float32),
            [pltpu.VMEM((CHE, A2W), jnp.float32) for _ in range(2)],
            [pltpu.VMEM((CHE, A2W), jnp.float32) for _ in range(2)],
            pltpu.VMEM_SHARED((NTP, A2W), jnp.float32),
            [pltpu.SemaphoreType.DMA for _ in range(2)],
            [pltpu.SemaphoreType.DMA for _ in range(2)],
        ],
    )
    def k(q_hbm, w_hbm, src_hbm, dst_hbm, out_hbm,
          srcv, dsca, wv, qg, pay, acc, gsems, ssems):
        cid = lax.axis_index("c")
        sid = lax.axis_index("s")
        wid = sid * 2 + cid
        row0 = wid * NCH
        lanes = lax.iota(jnp.int32, 16)

        _zero_acc(pay[0], acc, sid, A2W)
        plsc.subcore_barrier()

        def _fire(gnext, b):
            @pl.when(gnext < NCH)
            def _():
                pltpu.sync_copy(src_hbm.at[pl.ds(row0 + gnext, 1)], srcv[b])
                pltpu.async_copy(q_hbm.at[srcv[b].at[0]], qg[b], gsems[b])

        def _swait(b):
            pltpu.make_async_copy(pay[b], acc.at[dsca[b].at[0]], ssems[b]).wait()

        _fire(0, 0)

        def _half(g, b):
            @pl.when(g >= 2)
            def _():
                _swait(b)
            _fire(g + 1, 1 - b)
            pltpu.make_async_copy(q_hbm.at[srcv[b].at[0]], qg[b], gsems[b]).wait()
            ebase = (row0 + g) * 128
            pltpu.sync_copy(w_hbm.at[pl.ds(ebase, CHE)], wv)
            pltpu.sync_copy(dst_hbm.at[pl.ds(row0 + g, 1)], dsca[b])

            def _grp(j, _):
                el = j * 16 + lanes
                for h in range(H):
                    w = plsc.load_gather(wv, [el, _full16(h)])
                    for kk in range(DH):
                        col = _full16(h * DH + kk)
                        hv = plsc.load_gather(qg[b], [el, col])
                        plsc.store_scatter(pay[b], [el, col], w * hv)
                return _
            lax.fori_loop(0, CHE // 16, _grp, None)

            pltpu.async_copy(pay[b], acc.at[dsca[b].at[0]], ssems[b], add=True)

        def _pair(g0, _):
            _half(2 * g0, 0)
            _half(2 * g0 + 1, 1)
            return _
        lax.fori_loop(0, NCH // 2, _pair, None)

        _swait(0)
        _swait(1)
        plsc.subcore_barrier()
        pltpu.sync_copy(acc.at[pl.ds(sid * RPT, RPT)],
                        out_hbm.at[cid, pl.ds(sid * RPT, RPT)])

    return k(q, warr, src2d, dst2d)


def _blockdiag_scores(a):
    m = jnp.zeros((HID, H), jnp.float32)
    for h in range(H):
        m = m.at[h * DH:(h + 1) * DH, h].set(a[h])
    return m


def _combine_mats(we):
    m1 = jnp.zeros((A1W, HID), jnp.float32)
    m2 = jnp.zeros((A1W, HID), jnp.float32)
    for h in range(H):
        m1 = m1.at[4 + h * DE:4 + (h + 1) * DE, h * DH:(h + 1) * DH].set(
            we[:, h * DH:(h + 1) * DH])
        m2 = m2.at[h, h * DH:(h + 1) * DH].set(1.0)
    return m1, m2


def _eterm_coeff(we, a_e):
    c = jnp.stack([we[:, h * DH:(h + 1) * DH] @ a_e[h] for h in range(H)],
                  axis=1)
    return jnp.tile(c.reshape(16, 1), (1, 16))


def kernel(agent_id, bacth_nodes_feats, bacth_edge_index, bacth_edge_attr,
           rnn_states, masks, W1, We1, a_src1, a_dst1, a_e1, b1,
           W2, We2, a_src2, a_dst2, a_e2, b2,
           Wz, Uz, bz, Wr, Ur, br, Wn, Un, bn, Wv, bv):
    nodes = bacth_nodes_feats[:, 0].reshape(NT, DF)
    ei = bacth_edge_index[:, 0]
    eattr = bacth_edge_attr[:, 0].reshape(EC, DE)
    offs = (jnp.arange(B, dtype=jnp.int32) * N)[:, None]
    src = (ei[:, 0, :] + offs).reshape(EC)
    dst = (ei[:, 1, :] + offs).reshape(EC)
    src2d = jnp.pad(src, (0, EP - EC)).reshape(EP // 128, 128)
    dst2d = jnp.pad(dst, (0, EP - EC)).reshape(EP // 128, 128)
    eap = jnp.pad(eattr, ((0, EP - EC), (0, 0)))

    as1 = _blockdiag_scores(a_src1); ad1 = _blockdiag_scores(a_dst1)
    as2 = _blockdiag_scores(a_src2); ad2 = _blockdiag_scores(a_dst2)
    c1 = _eterm_coeff(We1, a_e1); c2 = _eterm_coeff(We2, a_e2)
    m1a, m2a = _combine_mats(We1)
    m1b, m2b = _combine_mats(We2)

    q1, s1, p1 = _make_qsp(nodes, W1, as1, ad1)
    parts1a, w1arr = _sc_logit_pass(s1, p1, eap, c1, src2d, dst2d)
    parts1b = _sc_message_pass(q1, w1arr, src2d, dst2d)
    q2, s2, p2 = _combine_layer(parts1b, parts1a, m1a, m2a, b1[None, :],
                                W2, as2, ad2)
    parts2a, w2arr = _sc_logit_pass(s2, p2, eap, c2, src2d, dst2d)
    parts2b = _sc_message_pass(q2, w2arr, src2d, dst2d)
    values, hnew = _head(parts2b, parts2a, m1b, m2b, b2[None, :],
                         rnn_states[:, 0], masks,
                         Wz, Uz, bz[None, :], Wr, Ur, br[None, :],
                         Wn, Un, bn[None, :], Wv, bv[None, :])
    return values, hnew[:, None, :]
